# baseline (speedup 1.0000x reference)
import functools

import jax
import jax.numpy as jnp
from jax import lax
from jax.experimental import pallas as pl
from jax.experimental.pallas import tpu as pltpu
from jax.experimental.pallas import tpu_sc as plsc

BATCH = 16384
HIST = 50
EMBED = 64
VOCAB = 1000000
TOTAL = BATCH * HIST

NUM_CORES = 2
NUM_SUBCORES = 16
NUM_WORKERS = NUM_CORES * NUM_SUBCORES

_mesh = plsc.VectorSubcoreMesh(core_axis_name="c", subcore_axis_name="s")


SPAN = 512
NBLK_FULL = VOCAB // SPAN
TAIL = VOCAB - NBLK_FULL * SPAN
BLK_PER_W = (NBLK_FULL + 1 + NUM_WORKERS - 1) // NUM_WORKERS


@functools.partial(
    pl.kernel,
    out_type=jax.ShapeDtypeStruct((VOCAB * EMBED,), jnp.float32),
    mesh=_mesh,
    scratch_types=[
        pltpu.VMEM((2, EMBED, SPAN + 1), jnp.float32),
        pltpu.VMEM((TAIL * EMBED,), jnp.float32),
        pltpu.VMEM((SPAN * EMBED,), jnp.float32),
        [pltpu.SemaphoreType.DMA] * 2,
        pltpu.SemaphoreType.DMA,
    ],
    compiler_params=pltpu.CompilerParams(
        use_tc_tiling_on_sc=True, needs_layout_passes=False
    ),
)
def _transpose_kernel(wt_hbm, tail_hbm, flat_hbm, ibuf, tailv, obuf, isems, wsem):
    wid = lax.axis_index("s") * NUM_CORES + lax.axis_index("c")

    def blk_of(t):
        return t * NUM_WORKERS + wid

    def issue_reads(blk, buf):
        for jj in range(8):
            pltpu.async_copy(
                wt_hbm.at[pl.ds(jj * 8, 8), pl.ds(blk * SPAN, SPAN)],
                ibuf.at[buf, pl.ds(jj * 8, 8), pl.ds(0, SPAN)],
                isems[buf],
            )

    def wait_reads(buf):
        for jj in range(8):
            pltpu.make_async_copy(
                wt_hbm.at[pl.ds(jj * 8, 8), pl.ds(0, SPAN)],
                ibuf.at[buf, pl.ds(jj * 8, 8), pl.ds(0, SPAN)],
                isems[buf],
            ).wait()

    def drain_write(n):
        pltpu.make_async_copy(
            obuf.at[pl.ds(0, n)], flat_hbm.at[pl.ds(0, n)], wsem
        ).wait()

    lane = lax.iota(jnp.int32, 16)
    perm_m = {st: (lane - st) % 16 for st in (8, 4, 2, 1)}
    perm_p = {st: (lane + st) % 16 for st in (8, 4, 2, 1)}
    masks = {st: (lane & st) == 0 for st in (8, 4, 2, 1)}

    def t16(r):
        for st in (8, 4, 2, 1):
            for i in range(16):
                if i & st:
                    continue
                a, b = r[i], r[i | st]
                bs = b.at[perm_m[st]].get(mode="promise_in_bounds")
                as_ = a.at[perm_p[st]].get(mode="promise_in_bounds")
                r[i] = jnp.where(masks[st], a, bs)
                r[i | st] = jnp.where(masks[st], as_, b)
        return r

    @pl.when(blk_of(0) < NBLK_FULL)
    def _():
        issue_reads(blk_of(0), 0)

    def body(p, nwr0):
        nwr = nwr0
        for buf in range(2):
            t = p * 2 + buf
            blk = blk_of(t)
            nxt = blk_of(t + 1)

            @pl.when(blk < NBLK_FULL)
            def _():
                wait_reads(buf)

                @pl.when(nxt < NBLK_FULL)
                def _():
                    issue_reads(nxt, 1 - buf)

                src = ibuf.at[buf]

                @pl.when(nwr > 0)
                def _():
                    drain_write(SPAN * EMBED)

                def tloop(i16, _):
                    i0 = i16 * 16
                    for j16 in range(4):
                        j0 = j16 * 16
                        r = [src[j0 + jl, pl.ds(i0, 16)] for jl in range(16)]
                        r = t16(r)
                        for il in range(16):
                            obuf[pl.ds((i0 + il) * EMBED + j0, 16)] = r[il]
                    return 0

                lax.fori_loop(0, SPAN // 16, tloop, 0)
                pltpu.async_copy(
                    obuf,
                    flat_hbm.at[pl.ds(blk * (SPAN * EMBED), SPAN * EMBED)],
                    wsem,
                )

            nwr = jnp.where(blk < NBLK_FULL, nwr + 1, nwr)
        return nwr

    nwr = lax.fori_loop(0, BLK_PER_W // 2, body, 0)

    @pl.when(wid == (NBLK_FULL % NUM_WORKERS))
    def _():
        pltpu.sync_copy(tail_hbm, tailv)
        pltpu.sync_copy(
            tailv, flat_hbm.at[pl.ds(NBLK_FULL * SPAN * EMBED, TAIL * EMBED)]
        )

    @pl.when(nwr > 0)
    def _():
        drain_write(SPAN * EMBED)


PER_WORKER = TOTAL // NUM_WORKERS
CHUNK = 320
NUM_CHUNKS = PER_WORKER // CHUNK
NBUF = 4
OUTER = NUM_CHUNKS // NBUF


@functools.partial(
    pl.kernel,
    out_type=jax.ShapeDtypeStruct((TOTAL, EMBED), jnp.float32),
    mesh=_mesh,
    scratch_types=[
        pltpu.VMEM((PER_WORKER,), jnp.int32),
        pltpu.VMEM((NBUF, CHUNK, EMBED), jnp.float32),
        [pltpu.SemaphoreType.DMA] * NBUF,
        [pltpu.SemaphoreType.DMA] * NBUF,
    ],
    compiler_params=pltpu.CompilerParams(use_tc_tiling_on_sc=False),
)
def _gather_kernel(weight_hbm, idx_hbm, out_hbm, idx_v, rows_v, gsems, wsems):
    wid = lax.axis_index("s") * NUM_CORES + lax.axis_index("c")
    base = wid * PER_WORKER
    pltpu.sync_copy(idx_hbm.at[pl.ds(base, PER_WORKER)], idx_v)

    def gather(g, b):
        pltpu.async_copy(
            weight_hbm.at[idx_v.at[pl.ds(g * CHUNK, CHUNK)]],
            rows_v.at[b],
            gsems[b],
        )

    def gather_wait(g, b):
        pltpu.make_async_copy(
            weight_hbm.at[idx_v.at[pl.ds(g * CHUNK, CHUNK)]],
            rows_v.at[b],
            gsems[b],
        ).wait()

    def writeback(g, b):
        pltpu.async_copy(
            rows_v.at[b], out_hbm.at[pl.ds(base + g * CHUNK, CHUNK)], wsems[b]
        )

    def writeback_wait(b):
        pltpu.make_async_copy(
            rows_v.at[b], out_hbm.at[pl.ds(base, CHUNK)], wsems[b]
        ).wait()

    gather(0, 0)
    gather(1, 1)

    def body(p, _):
        for j in range(NBUF):
            g = p * NBUF + j
            h = g + 2
            gather_wait(g, j)

            @pl.when(h < NUM_CHUNKS)
            def _():
                bh = (j + 2) % NBUF

                @pl.when(g >= 2)
                def _():
                    writeback_wait(bh)

                gather(h, bh)

            writeback(g, j)
        return 0

    lax.fori_loop(0, OUTER, body, 0)
    writeback_wait((NUM_CHUNKS - 2) % NBUF)
    writeback_wait((NUM_CHUNKS - 1) % NBUF)


def kernel(input, weight):
    tail = weight[NBLK_FULL * SPAN :, :].reshape(TAIL * EMBED)
    flat = _transpose_kernel(weight.T, tail)
    table = flat.reshape(VOCAB, EMBED)
    idx = input.astype(jnp.int32).reshape(TOTAL)
    out = _gather_kernel(table, idx)
    return out.reshape(BATCH, HIST, EMBED)


# speedup vs baseline: 1.5363x; 1.5363x over previous
import functools

import jax
import jax.numpy as jnp
from jax import lax
from jax.experimental import pallas as pl
from jax.experimental.pallas import tpu as pltpu
from jax.experimental.pallas import tpu_sc as plsc

BATCH = 16384
HIST = 50
EMBED = 64
VOCAB = 1000000
TOTAL = BATCH * HIST

NUM_CORES = 2
NUM_SUBCORES = 16
NUM_WORKERS = NUM_CORES * NUM_SUBCORES

_mesh = plsc.VectorSubcoreMesh(core_axis_name="c", subcore_axis_name="s")


SPAN = 512
NBLK_FULL = VOCAB // SPAN
TAIL = VOCAB - NBLK_FULL * SPAN
BLK_PER_W = (NBLK_FULL + 1 + NUM_WORKERS - 1) // NUM_WORKERS


@functools.partial(
    pl.kernel,
    out_type=jax.ShapeDtypeStruct((VOCAB * EMBED,), jnp.float32),
    mesh=_mesh,
    scratch_types=[
        pltpu.VMEM((2, EMBED, SPAN + 1), jnp.float32),
        pltpu.VMEM((TAIL * EMBED,), jnp.float32),
        pltpu.VMEM((SPAN * EMBED,), jnp.float32),
        [pltpu.SemaphoreType.DMA] * 2,
        pltpu.SemaphoreType.DMA,
    ],
    compiler_params=pltpu.CompilerParams(
        use_tc_tiling_on_sc=True, needs_layout_passes=False
    ),
)
def _transpose_kernel(wt_hbm, tail_hbm, flat_hbm, ibuf, tailv, obuf, isems, wsem):
    wid = lax.axis_index("s") * NUM_CORES + lax.axis_index("c")

    def blk_of(t):
        return t * NUM_WORKERS + wid

    def issue_reads(blk, buf):
        for jj in range(8):
            pltpu.async_copy(
                wt_hbm.at[pl.ds(jj * 8, 8), pl.ds(blk * SPAN, SPAN)],
                ibuf.at[buf, pl.ds(jj * 8, 8), pl.ds(0, SPAN)],
                isems[buf],
            )

    def wait_reads(buf):
        for jj in range(8):
            pltpu.make_async_copy(
                wt_hbm.at[pl.ds(jj * 8, 8), pl.ds(0, SPAN)],
                ibuf.at[buf, pl.ds(jj * 8, 8), pl.ds(0, SPAN)],
                isems[buf],
            ).wait()

    def drain_write(n):
        pltpu.make_async_copy(
            obuf.at[pl.ds(0, n)], flat_hbm.at[pl.ds(0, n)], wsem
        ).wait()

    lane = lax.iota(jnp.int32, 16)
    perm_m = {st: (lane - st) % 16 for st in (8, 4, 2, 1)}
    perm_p = {st: (lane + st) % 16 for st in (8, 4, 2, 1)}
    masks = {st: (lane & st) == 0 for st in (8, 4, 2, 1)}

    def t16(r):
        for st in (8, 4, 2, 1):
            for i in range(16):
                if i & st:
                    continue
                a, b = r[i], r[i | st]
                bs = b.at[perm_m[st]].get(mode="promise_in_bounds")
                as_ = a.at[perm_p[st]].get(mode="promise_in_bounds")
                r[i] = jnp.where(masks[st], a, bs)
                r[i | st] = jnp.where(masks[st], as_, b)
        return r

    @pl.when(blk_of(0) < NBLK_FULL)
    def _():
        issue_reads(blk_of(0), 0)

    def body(p, nwr0):
        nwr = nwr0
        for buf in range(2):
            t = p * 2 + buf
            blk = blk_of(t)
            nxt = blk_of(t + 1)

            @pl.when(blk < NBLK_FULL)
            def _():
                wait_reads(buf)

                @pl.when(nxt < NBLK_FULL)
                def _():
                    issue_reads(nxt, 1 - buf)

                src = ibuf.at[buf]

                @pl.when(nwr > 0)
                def _():
                    drain_write(SPAN * EMBED)

                def tloop(i16, _):
                    i0 = i16 * 16
                    for j16 in range(4):
                        j0 = j16 * 16
                        r = [src[j0 + jl, pl.ds(i0, 16)] for jl in range(16)]
                        r = t16(r)
                        for il in range(16):
                            obuf[pl.ds((i0 + il) * EMBED + j0, 16)] = r[il]
                    return 0

                lax.fori_loop(0, SPAN // 16, tloop, 0)
                pltpu.async_copy(
                    obuf,
                    flat_hbm.at[pl.ds(blk * (SPAN * EMBED), SPAN * EMBED)],
                    wsem,
                )

            nwr = jnp.where(blk < NBLK_FULL, nwr + 1, nwr)
        return nwr

    nwr = lax.fori_loop(0, BLK_PER_W // 2, body, 0)

    @pl.when(wid == (NBLK_FULL % NUM_WORKERS))
    def _():
        pltpu.sync_copy(tail_hbm, tailv)
        pltpu.sync_copy(
            tailv, flat_hbm.at[pl.ds(NBLK_FULL * SPAN * EMBED, TAIL * EMBED)]
        )

    @pl.when(nwr > 0)
    def _():
        drain_write(SPAN * EMBED)


CHUNK2 = 512
NBC = BATCH // CHUNK2
NTASK = HIST * NBC
TASK_PER_W = NTASK // NUM_WORKERS


@functools.partial(
    pl.kernel,
    out_type=jax.ShapeDtypeStruct((HIST, 8, BATCH // 128, 8, 128), jnp.float32),
    mesh=_mesh,
    scratch_types=[
        pltpu.VMEM((2, CHUNK2), jnp.int32),
        pltpu.VMEM((2, CHUNK2, EMBED), jnp.float32),
        pltpu.VMEM((8, CHUNK2 // 128, 8, 128), jnp.float32),
        [pltpu.SemaphoreType.DMA] * 2,
        [pltpu.SemaphoreType.DMA] * 2,
        pltpu.SemaphoreType.DMA,
    ],
    compiler_params=pltpu.CompilerParams(use_tc_tiling_on_sc=False),
)
def _gather_kernel(table_hbm, idx_hbm, out_hbm, idx_v, rows_v, obuf, isems, gsems, wsem):
    wid = lax.axis_index("s") * NUM_CORES + lax.axis_index("c")
    tbase = wid * TASK_PER_W

    def task_hb(t):
        T = tbase + t
        return T // NBC, T % NBC

    lane = lax.iota(jnp.int32, 16)
    perm_m = {st: (lane - st) % 16 for st in (8, 4, 2, 1)}
    perm_p = {st: (lane + st) % 16 for st in (8, 4, 2, 1)}
    masks = {st: (lane & st) == 0 for st in (8, 4, 2, 1)}

    def t16(r):
        for st in (8, 4, 2, 1):
            for i in range(16):
                if i & st:
                    continue
                a, b = r[i], r[i | st]
                bs = b.at[perm_m[st]].get(mode="promise_in_bounds")
                as_ = a.at[perm_p[st]].get(mode="promise_in_bounds")
                r[i] = jnp.where(masks[st], a, bs)
                r[i | st] = jnp.where(masks[st], as_, b)
        return r

    def idx_load(t, buf, sem):
        h, bc = task_hb(t)
        pltpu.async_copy(
            idx_hbm.at[h, pl.ds(bc * CHUNK2, CHUNK2)], idx_v.at[buf], sem
        )

    def idx_wait(buf, sem):
        pltpu.make_async_copy(
            idx_hbm.at[0, pl.ds(0, CHUNK2)], idx_v.at[buf], sem
        ).wait()

    def gather(buf):
        pltpu.async_copy(
            table_hbm.at[idx_v.at[buf]], rows_v.at[buf], gsems[buf]
        )

    def gather_wait(buf):
        pltpu.make_async_copy(
            table_hbm.at[idx_v.at[buf]], rows_v.at[buf], gsems[buf]
        ).wait()

    def writes(t):
        h, bc = task_hb(t)
        for jj in range(8):
            pltpu.async_copy(
                obuf.at[jj],
                out_hbm.at[h, jj, pl.ds(bc * (CHUNK2 // 128), CHUNK2 // 128)],
                wsem,
            )

    def writes_drain():
        for jj in range(8):
            pltpu.make_async_copy(
                obuf.at[jj], out_hbm.at[0, jj, pl.ds(0, CHUNK2 // 128)], wsem
            ).wait()

    idx_load(0, 0, isems[0])
    idx_wait(0, isems[0])
    gather(0)
    idx_load(1, 1, isems[1])

    def body(p, _):
        for buf in range(2):
            t = p * 2 + buf
            nbuf = 1 - buf
            gather_wait(buf)

            @pl.when(t + 1 < TASK_PER_W)
            def _():
                idx_wait(nbuf, isems[nbuf])
                gather(nbuf)

            @pl.when(t + 2 < TASK_PER_W)
            def _():
                idx_load(t + 2, buf, isems[buf])

            @pl.when(t > 0)
            def _():
                writes_drain()

            src = rows_v.at[buf]

            def tloop(bq, _):
                b0 = bq * 16
                bb = bq // 8
                brq = (bq % 8) * 16
                for j16 in range(4):
                    j0 = j16 * 16
                    r = [src[b0 + bl, pl.ds(j0, 16)] for bl in range(16)]
                    r = t16(r)
                    for jl in range(16):
                        jj = j16 * 2 + jl // 8
                        jr = jl % 8
                        obuf[jj, bb, jr, pl.ds(brq, 16)] = r[jl]
                return 0

            lax.fori_loop(0, CHUNK2 // 16, tloop, 0)
            writes(t)
        return 0

    lax.fori_loop(0, TASK_PER_W // 2, body, 0)
    writes_drain()


def kernel(input, weight):
    tail = weight[NBLK_FULL * SPAN :, :].reshape(TAIL * EMBED)
    flat = _transpose_kernel(weight.T, tail)
    table = flat.reshape(VOCAB, EMBED)
    idx_t = input.astype(jnp.int32).T
    out5 = _gather_kernel(table, idx_t)
    return out5.transpose(2, 4, 0, 1, 3).reshape(BATCH, HIST, EMBED)


# speedup vs baseline: 1.6843x; 1.0963x over previous
import functools

import jax
import jax.numpy as jnp
from jax import lax
from jax.experimental import pallas as pl
from jax.experimental.pallas import tpu as pltpu
from jax.experimental.pallas import tpu_sc as plsc

BATCH = 16384
HIST = 50
EMBED = 64
VOCAB = 1000000
TOTAL = BATCH * HIST

NUM_CORES = 2
NUM_SUBCORES = 16
NUM_WORKERS = NUM_CORES * NUM_SUBCORES

_mesh = plsc.VectorSubcoreMesh(core_axis_name="c", subcore_axis_name="s")


SPAN = 512
NBLK_FULL = VOCAB // SPAN
TAIL = VOCAB - NBLK_FULL * SPAN
BLK_PER_W = (NBLK_FULL + 1 + NUM_WORKERS - 1) // NUM_WORKERS


@functools.partial(
    pl.kernel,
    out_type=jax.ShapeDtypeStruct((VOCAB * EMBED,), jnp.float32),
    mesh=_mesh,
    scratch_types=[
        pltpu.VMEM((2, EMBED, SPAN + 1), jnp.float32),
        pltpu.VMEM((TAIL * EMBED,), jnp.float32),
        pltpu.VMEM((SPAN * EMBED,), jnp.float32),
        [pltpu.SemaphoreType.DMA] * 2,
        pltpu.SemaphoreType.DMA,
    ],
    compiler_params=pltpu.CompilerParams(
        use_tc_tiling_on_sc=True, needs_layout_passes=False
    ),
)
def _transpose_kernel(wt_hbm, tail_hbm, flat_hbm, ibuf, tailv, obuf, isems, wsem):
    wid = lax.axis_index("s") * NUM_CORES + lax.axis_index("c")

    def blk_of(t):
        return t * NUM_WORKERS + wid

    def issue_reads(blk, buf):
        for jj in range(8):
            pltpu.async_copy(
                wt_hbm.at[pl.ds(jj * 8, 8), pl.ds(blk * SPAN, SPAN)],
                ibuf.at[buf, pl.ds(jj * 8, 8), pl.ds(0, SPAN)],
                isems[buf],
            )

    def wait_reads(buf):
        for jj in range(8):
            pltpu.make_async_copy(
                wt_hbm.at[pl.ds(jj * 8, 8), pl.ds(0, SPAN)],
                ibuf.at[buf, pl.ds(jj * 8, 8), pl.ds(0, SPAN)],
                isems[buf],
            ).wait()

    def drain_write(n):
        pltpu.make_async_copy(
            obuf.at[pl.ds(0, n)], flat_hbm.at[pl.ds(0, n)], wsem
        ).wait()

    lane = lax.iota(jnp.int32, 16)
    perm_m = {st: (lane - st) % 16 for st in (8, 4, 2, 1)}
    perm_p = {st: (lane + st) % 16 for st in (8, 4, 2, 1)}
    masks = {st: (lane & st) == 0 for st in (8, 4, 2, 1)}

    def t16(r):
        for st in (8, 4, 2, 1):
            for i in range(16):
                if i & st:
                    continue
                a, b = r[i], r[i | st]
                bs = b.at[perm_m[st]].get(mode="promise_in_bounds")
                as_ = a.at[perm_p[st]].get(mode="promise_in_bounds")
                r[i] = jnp.where(masks[st], a, bs)
                r[i | st] = jnp.where(masks[st], as_, b)
        return r

    @pl.when(blk_of(0) < NBLK_FULL)
    def _():
        issue_reads(blk_of(0), 0)

    def body(p, nwr0):
        nwr = nwr0
        for buf in range(2):
            t = p * 2 + buf
            blk = blk_of(t)
            nxt = blk_of(t + 1)

            @pl.when(blk < NBLK_FULL)
            def _():
                wait_reads(buf)

                @pl.when(nxt < NBLK_FULL)
                def _():
                    issue_reads(nxt, 1 - buf)

                src = ibuf.at[buf]

                @pl.when(nwr > 0)
                def _():
                    drain_write(SPAN * EMBED)

                def tloop(i16, _):
                    i0 = i16 * 16
                    for j16 in range(4):
                        j0 = j16 * 16
                        r = [src[j0 + jl, pl.ds(i0, 16)] for jl in range(16)]
                        r = t16(r)
                        for il in range(16):
                            obuf[pl.ds((i0 + il) * EMBED + j0, 16)] = r[il]
                    return 0

                lax.fori_loop(0, SPAN // 16, tloop, 0)
                pltpu.async_copy(
                    obuf,
                    flat_hbm.at[pl.ds(blk * (SPAN * EMBED), SPAN * EMBED)],
                    wsem,
                )

            nwr = jnp.where(blk < NBLK_FULL, nwr + 1, nwr)
        return nwr

    nwr = lax.fori_loop(0, BLK_PER_W // 2, body, 0)

    @pl.when(wid == (NBLK_FULL % NUM_WORKERS))
    def _():
        pltpu.sync_copy(tail_hbm, tailv)
        pltpu.sync_copy(
            tailv, flat_hbm.at[pl.ds(NBLK_FULL * SPAN * EMBED, TAIL * EMBED)]
        )

    @pl.when(nwr > 0)
    def _():
        drain_write(SPAN * EMBED)


CHUNK2 = 256
NBC = BATCH // CHUNK2
NTASK = HIST * NBC
TASK_PER_W = NTASK // NUM_WORKERS


@functools.partial(
    pl.kernel,
    out_type=jax.ShapeDtypeStruct((HIST, 8, BATCH // 128, 8, 128), jnp.float32),
    mesh=_mesh,
    scratch_types=[
        pltpu.VMEM((2, CHUNK2), jnp.int32),
        pltpu.VMEM((2, CHUNK2, EMBED), jnp.float32),
        pltpu.VMEM((2, 8, CHUNK2 // 128, 8, 128), jnp.float32),
        [pltpu.SemaphoreType.DMA] * 2,
        [pltpu.SemaphoreType.DMA] * 2,
        pltpu.SemaphoreType.DMA,
    ],
    compiler_params=pltpu.CompilerParams(use_tc_tiling_on_sc=False),
)
def _gather_kernel(table_hbm, idx_hbm, out_hbm, idx_v, rows_v, obuf, isems, gsems, wsem):
    wid = lax.axis_index("s") * NUM_CORES + lax.axis_index("c")
    tbase = wid * TASK_PER_W

    def task_hb(t):
        T = tbase + t
        return T // NBC, T % NBC

    lane = lax.iota(jnp.int32, 16)
    perm_m = {st: (lane - st) % 16 for st in (8, 4, 2, 1)}
    perm_p = {st: (lane + st) % 16 for st in (8, 4, 2, 1)}
    masks = {st: (lane & st) == 0 for st in (8, 4, 2, 1)}

    def t16(r):
        for st in (8, 4, 2, 1):
            for i in range(16):
                if i & st:
                    continue
                a, b = r[i], r[i | st]
                bs = b.at[perm_m[st]].get(mode="promise_in_bounds")
                as_ = a.at[perm_p[st]].get(mode="promise_in_bounds")
                r[i] = jnp.where(masks[st], a, bs)
                r[i | st] = jnp.where(masks[st], as_, b)
        return r

    def idx_load(t, buf, sem):
        h, bc = task_hb(t)
        pltpu.async_copy(
            idx_hbm.at[h, pl.ds(bc * CHUNK2, CHUNK2)], idx_v.at[buf], sem
        )

    def idx_wait(buf, sem):
        pltpu.make_async_copy(
            idx_hbm.at[0, pl.ds(0, CHUNK2)], idx_v.at[buf], sem
        ).wait()

    def gather(buf):
        pltpu.async_copy(
            table_hbm.at[idx_v.at[buf]], rows_v.at[buf], gsems[buf]
        )

    def gather_wait(buf):
        pltpu.make_async_copy(
            table_hbm.at[idx_v.at[buf]], rows_v.at[buf], gsems[buf]
        ).wait()

    def writes(t, ob):
        h, bc = task_hb(t)
        for jj in range(8):
            pltpu.async_copy(
                obuf.at[ob, jj],
                out_hbm.at[h, jj, pl.ds(bc * (CHUNK2 // 128), CHUNK2 // 128)],
                wsem,
            )

    def writes_drain(ob):
        for jj in range(8):
            pltpu.make_async_copy(
                obuf.at[ob, jj], out_hbm.at[0, jj, pl.ds(0, CHUNK2 // 128)], wsem
            ).wait()

    idx_load(0, 0, isems[0])
    idx_wait(0, isems[0])
    gather(0)
    idx_load(1, 1, isems[1])

    def body(p, _):
        for buf in range(2):
            t = p * 2 + buf
            nbuf = 1 - buf
            gather_wait(buf)

            @pl.when(t + 1 < TASK_PER_W)
            def _():
                idx_wait(nbuf, isems[nbuf])
                gather(nbuf)

            @pl.when(t + 2 < TASK_PER_W)
            def _():
                idx_load(t + 2, buf, isems[buf])

            @pl.when(t > 1)
            def _():
                writes_drain(buf)

            src = rows_v.at[buf]

            def tloop(bq, _):
                b0 = bq * 16
                bb = bq // 8
                brq = (bq % 8) * 16
                for j16 in range(4):
                    j0 = j16 * 16
                    r = [src[b0 + bl, pl.ds(j0, 16)] for bl in range(16)]
                    r = t16(r)
                    for jl in range(16):
                        jj = j16 * 2 + jl // 8
                        jr = jl % 8
                        obuf[buf, jj, bb, jr, pl.ds(brq, 16)] = r[jl]
                return 0

            lax.fori_loop(0, CHUNK2 // 16, tloop, 0)
            writes(t, buf)
        return 0

    lax.fori_loop(0, TASK_PER_W // 2, body, 0)
    writes_drain(0)
    writes_drain(1)


def kernel(input, weight):
    tail = weight[NBLK_FULL * SPAN :, :].reshape(TAIL * EMBED)
    flat = _transpose_kernel(weight.T, tail)
    table = flat.reshape(VOCAB, EMBED)
    idx_t = input.astype(jnp.int32).T
    out5 = _gather_kernel(table, idx_t)
    return out5.transpose(2, 4, 0, 1, 3).reshape(BATCH, HIST, EMBED)


# speedup vs baseline: 1.7026x; 1.0109x over previous
import functools

import jax
import jax.numpy as jnp
from jax import lax
from jax.experimental import pallas as pl
from jax.experimental.pallas import tpu as pltpu
from jax.experimental.pallas import tpu_sc as plsc

BATCH = 16384
HIST = 50
EMBED = 64
VOCAB = 1000000
TOTAL = BATCH * HIST

NUM_CORES = 2
NUM_SUBCORES = 16
NUM_WORKERS = NUM_CORES * NUM_SUBCORES

_mesh = plsc.VectorSubcoreMesh(core_axis_name="c", subcore_axis_name="s")


SPAN = 256
NBLK_FULL = VOCAB // SPAN
TAIL = VOCAB - NBLK_FULL * SPAN
BLK_PER_W = -(-(NBLK_FULL + 1) // NUM_WORKERS) * 2 // 2
BLK_PER_W += BLK_PER_W % 2


@functools.partial(
    pl.kernel,
    out_type=jax.ShapeDtypeStruct((VOCAB * EMBED,), jnp.float32),
    mesh=_mesh,
    scratch_types=[
        pltpu.VMEM((2, EMBED, SPAN + 1), jnp.float32),
        pltpu.VMEM((TAIL * EMBED,), jnp.float32),
        pltpu.VMEM((2, SPAN * EMBED), jnp.float32),
        [pltpu.SemaphoreType.DMA] * 2,
        pltpu.SemaphoreType.DMA,
    ],
    compiler_params=pltpu.CompilerParams(
        use_tc_tiling_on_sc=True, needs_layout_passes=False
    ),
)
def _transpose_kernel(wt_hbm, tail_hbm, flat_hbm, ibuf, tailv, obuf, isems, wsem):
    wid = lax.axis_index("s") * NUM_CORES + lax.axis_index("c")

    def blk_of(t):
        return t * NUM_WORKERS + wid

    def issue_reads(blk, buf):
        for jj in range(8):
            pltpu.async_copy(
                wt_hbm.at[pl.ds(jj * 8, 8), pl.ds(blk * SPAN, SPAN)],
                ibuf.at[buf, pl.ds(jj * 8, 8), pl.ds(0, SPAN)],
                isems[buf],
            )

    def wait_reads(buf):
        for jj in range(8):
            pltpu.make_async_copy(
                wt_hbm.at[pl.ds(jj * 8, 8), pl.ds(0, SPAN)],
                ibuf.at[buf, pl.ds(jj * 8, 8), pl.ds(0, SPAN)],
                isems[buf],
            ).wait()

    def drain_write(ob, n):
        pltpu.make_async_copy(
            obuf.at[ob, pl.ds(0, n)], flat_hbm.at[pl.ds(0, n)], wsem
        ).wait()

    lane = lax.iota(jnp.int32, 16)
    perm_m = {st: (lane - st) % 16 for st in (8, 4, 2, 1)}
    perm_p = {st: (lane + st) % 16 for st in (8, 4, 2, 1)}
    masks = {st: (lane & st) == 0 for st in (8, 4, 2, 1)}

    def t16(r):
        for st in (8, 4, 2, 1):
            for i in range(16):
                if i & st:
                    continue
                a, b = r[i], r[i | st]
                bs = b.at[perm_m[st]].get(mode="promise_in_bounds")
                as_ = a.at[perm_p[st]].get(mode="promise_in_bounds")
                r[i] = jnp.where(masks[st], a, bs)
                r[i | st] = jnp.where(masks[st], as_, b)
        return r

    @pl.when(blk_of(0) < NBLK_FULL)
    def _():
        issue_reads(blk_of(0), 0)

    def body(p, nwr0):
        nwr = nwr0
        for buf in range(2):
            t = p * 2 + buf
            blk = blk_of(t)
            nxt = blk_of(t + 1)

            @pl.when(blk < NBLK_FULL)
            def _():
                wait_reads(buf)

                @pl.when(nxt < NBLK_FULL)
                def _():
                    issue_reads(nxt, 1 - buf)

                src = ibuf.at[buf]

                @pl.when(nwr > 1)
                def _():
                    drain_write(buf, SPAN * EMBED)

                def tloop(i16, _):
                    i0 = i16 * 16
                    for j16 in range(4):
                        j0 = j16 * 16
                        r = [src[j0 + jl, pl.ds(i0, 16)] for jl in range(16)]
                        r = t16(r)
                        for il in range(16):
                            obuf[buf, pl.ds((i0 + il) * EMBED + j0, 16)] = r[il]
                    return 0

                lax.fori_loop(0, SPAN // 16, tloop, 0)
                pltpu.async_copy(
                    obuf.at[buf],
                    flat_hbm.at[pl.ds(blk * (SPAN * EMBED), SPAN * EMBED)],
                    wsem,
                )

            nwr = jnp.where(blk < NBLK_FULL, nwr + 1, nwr)
        return nwr

    nwr = lax.fori_loop(0, BLK_PER_W // 2, body, 0)

    @pl.when(wid == (NBLK_FULL % NUM_WORKERS))
    def _():
        pltpu.sync_copy(tail_hbm, tailv)
        pltpu.sync_copy(
            tailv, flat_hbm.at[pl.ds(NBLK_FULL * SPAN * EMBED, TAIL * EMBED)]
        )

    @pl.when(nwr > 1)
    def _():
        drain_write(0, SPAN * EMBED)

    @pl.when(nwr > 0)
    def _():
        drain_write(1 - (nwr % 2), SPAN * EMBED)


CHUNK2 = 256
NBC = BATCH // CHUNK2
NTASK = HIST * NBC
TASK_PER_W = NTASK // NUM_WORKERS


@functools.partial(
    pl.kernel,
    out_type=jax.ShapeDtypeStruct((HIST, 8, BATCH // 128, 8, 128), jnp.float32),
    mesh=_mesh,
    scratch_types=[
        pltpu.VMEM((2, CHUNK2), jnp.int32),
        pltpu.VMEM((2, CHUNK2, EMBED), jnp.float32),
        pltpu.VMEM((2, 8, CHUNK2 // 128, 8, 128), jnp.float32),
        [pltpu.SemaphoreType.DMA] * 2,
        [pltpu.SemaphoreType.DMA] * 2,
        pltpu.SemaphoreType.DMA,
    ],
    compiler_params=pltpu.CompilerParams(use_tc_tiling_on_sc=False),
)
def _gather_kernel(table_hbm, idx_hbm, out_hbm, idx_v, rows_v, obuf, isems, gsems, wsem):
    wid = lax.axis_index("s") * NUM_CORES + lax.axis_index("c")
    tbase = wid * TASK_PER_W

    def task_hb(t):
        T = tbase + t
        return T // NBC, T % NBC

    lane = lax.iota(jnp.int32, 16)
    perm_m = {st: (lane - st) % 16 for st in (8, 4, 2, 1)}
    perm_p = {st: (lane + st) % 16 for st in (8, 4, 2, 1)}
    masks = {st: (lane & st) == 0 for st in (8, 4, 2, 1)}

    def t16(r):
        for st in (8, 4, 2, 1):
            for i in range(16):
                if i & st:
                    continue
                a, b = r[i], r[i | st]
                bs = b.at[perm_m[st]].get(mode="promise_in_bounds")
                as_ = a.at[perm_p[st]].get(mode="promise_in_bounds")
                r[i] = jnp.where(masks[st], a, bs)
                r[i | st] = jnp.where(masks[st], as_, b)
        return r

    def idx_load(t, buf, sem):
        h, bc = task_hb(t)
        pltpu.async_copy(
            idx_hbm.at[h, pl.ds(bc * CHUNK2, CHUNK2)], idx_v.at[buf], sem
        )

    def idx_wait(buf, sem):
        pltpu.make_async_copy(
            idx_hbm.at[0, pl.ds(0, CHUNK2)], idx_v.at[buf], sem
        ).wait()

    def gather(buf):
        pltpu.async_copy(
            table_hbm.at[idx_v.at[buf]], rows_v.at[buf], gsems[buf]
        )

    def gather_wait(buf):
        pltpu.make_async_copy(
            table_hbm.at[idx_v.at[buf]], rows_v.at[buf], gsems[buf]
        ).wait()

    def writes(t, ob):
        h, bc = task_hb(t)
        for jj in range(8):
            pltpu.async_copy(
                obuf.at[ob, jj],
                out_hbm.at[h, jj, pl.ds(bc * (CHUNK2 // 128), CHUNK2 // 128)],
                wsem,
            )

    def writes_drain(ob):
        for jj in range(8):
            pltpu.make_async_copy(
                obuf.at[ob, jj], out_hbm.at[0, jj, pl.ds(0, CHUNK2 // 128)], wsem
            ).wait()

    idx_load(0, 0, isems[0])
    idx_wait(0, isems[0])
    gather(0)
    idx_load(1, 1, isems[1])

    def body(p, _):
        for buf in range(2):
            t = p * 2 + buf
            nbuf = 1 - buf
            gather_wait(buf)

            @pl.when(t + 1 < TASK_PER_W)
            def _():
                idx_wait(nbuf, isems[nbuf])
                gather(nbuf)

            @pl.when(t + 2 < TASK_PER_W)
            def _():
                idx_load(t + 2, buf, isems[buf])

            @pl.when(t > 1)
            def _():
                writes_drain(buf)

            src = rows_v.at[buf]

            def tloop(bq, _):
                b0 = bq * 16
                bb = bq // 8
                brq = (bq % 8) * 16
                for j16 in range(4):
                    j0 = j16 * 16
                    r = [src[b0 + bl, pl.ds(j0, 16)] for bl in range(16)]
                    r = t16(r)
                    for jl in range(16):
                        jj = j16 * 2 + jl // 8
                        jr = jl % 8
                        obuf[buf, jj, bb, jr, pl.ds(brq, 16)] = r[jl]
                return 0

            lax.fori_loop(0, CHUNK2 // 16, tloop, 0)
            writes(t, buf)
        return 0

    lax.fori_loop(0, TASK_PER_W // 2, body, 0)
    writes_drain(0)
    writes_drain(1)


def kernel(input, weight):
    tail = weight[NBLK_FULL * SPAN :, :].reshape(TAIL * EMBED)
    flat = _transpose_kernel(weight.T, tail)
    table = flat.reshape(VOCAB, EMBED)
    idx_t = input.astype(jnp.int32).T
    out5 = _gather_kernel(table, idx_t)
    return out5.transpose(2, 4, 0, 1, 3).reshape(BATCH, HIST, EMBED)
